# baseline (speedup 1.0000x reference)
import jax
import jax.numpy as jnp
from jax import lax
from jax.experimental import pallas as pl
from jax.experimental.pallas import tpu as pltpu
from jax.experimental.pallas import tpu_sc as plsc

_N = 10000
_NPAD = 10240
_E = 320000
_D = 128
_H = 32
_CH = 128
_NWORK = 32
_CPT = 80
_EPAD = _NWORK * _CPT * _CH
_ROWS_PER_TILE = _NPAD // 16
_NBUF = 6
_GLOOK = 4
_SLAG = _NBUF - _GLOOK

_mesh = plsc.VectorSubcoreMesh(core_axis_name="c", subcore_axis_name="s")


def _sc_degree_body(dst_ref, out_ref, dstv, hist):
    c = lax.axis_index("c")
    s = lax.axis_index("s")
    wid = s * 2 + c
    z16 = jnp.zeros((16,), jnp.float32)

    def zero_body(i, carry):
        hist[pl.ds(i * 16, 16)] = z16
        return carry

    lax.fori_loop(0, _NPAD // 16, zero_body, 0)

    pltpu.sync_copy(dst_ref.at[pl.ds(wid * _CPT, _CPT)], dstv)

    ones = jnp.full((16,), 1.0, jnp.float32)

    def chunk_body(i, carry):
        for j in range(_CH // 16):
            idx = dstv[i, pl.ds(j * 16, 16)]
            plsc.addupdate_scatter(hist, [idx], ones)
        return carry

    lax.fori_loop(0, _CPT, chunk_body, 0)
    pltpu.sync_copy(hist, out_ref.at[wid])


_sc_degree = pl.kernel(
    _sc_degree_body,
    out_type=jax.ShapeDtypeStruct((_NWORK, _NPAD), jnp.float32),
    mesh=_mesh,
    scratch_types=[
        pltpu.VMEM((_CPT, _CH), jnp.int32),
        pltpu.VMEM((_NPAD,), jnp.float32),
    ],
    compiler_params=pltpu.CompilerParams(needs_layout_passes=False),
)


def _sc_conv_body(
    ht_ref, src_ref, dst_ref, out_ref, acc, srcv, dstv, gbuf, zbuf, gsem, ssem
):
    c = lax.axis_index("c")
    s = lax.axis_index("s")
    wid = s * 2 + c
    z16 = jnp.zeros((16,), jnp.float32)

    def zb(i, carry):
        zbuf[i, pl.ds(0, 16)] = z16
        zbuf[i, pl.ds(16, 16)] = z16
        return carry

    lax.fori_loop(0, _CH, zb, 0)
    for k in range(_ROWS_PER_TILE // _CH):
        pltpu.sync_copy(zbuf, acc.at[pl.ds(s * _ROWS_PER_TILE + k * _CH, _CH)])

    pltpu.sync_copy(src_ref.at[pl.ds(wid * _CPT, _CPT)], srcv)
    pltpu.sync_copy(dst_ref.at[pl.ds(wid * _CPT, _CPT)], dstv)
    plsc.subcore_barrier()

    for b in range(_GLOOK):
        pltpu.async_copy(ht_ref.at[srcv.at[b]], gbuf.at[b], gsem)

    def chunk(i, carry):
        buf = lax.rem(i, _NBUF)
        pltpu.make_async_copy(ht_ref.at[srcv.at[i]], gbuf.at[buf], gsem).wait()
        pltpu.async_copy(gbuf.at[buf], acc.at[dstv.at[i]], ssem, add=True)

        @pl.when(i >= _SLAG)
        def _():
            j = i - _SLAG
            jb = lax.rem(j, _NBUF)
            pltpu.make_async_copy(gbuf.at[jb], acc.at[dstv.at[j]], ssem).wait()

        @pl.when(i + _GLOOK < _CPT)
        def _():
            k = i + _GLOOK
            pltpu.async_copy(ht_ref.at[srcv.at[k]], gbuf.at[lax.rem(k, _NBUF)], gsem)

        return carry

    lax.fori_loop(0, _CPT, chunk, 0)
    for j in range(_CPT - _SLAG, _CPT):
        pltpu.make_async_copy(
            gbuf.at[j % _NBUF], acc.at[dstv.at[j]], ssem
        ).wait()
    plsc.subcore_barrier()
    pltpu.sync_copy(
        acc.at[pl.ds(s * _ROWS_PER_TILE, _ROWS_PER_TILE)],
        out_ref.at[c, pl.ds(s * _ROWS_PER_TILE, _ROWS_PER_TILE)],
    )


_sc_conv = pl.kernel(
    _sc_conv_body,
    out_type=jax.ShapeDtypeStruct((2, _NPAD, _H), jnp.float32),
    mesh=_mesh,
    scratch_types=[
        pltpu.VMEM_SHARED((_NPAD, _H), jnp.float32),
        pltpu.VMEM((_CPT, _CH), jnp.int32),
        pltpu.VMEM((_CPT, _CH), jnp.int32),
        pltpu.VMEM((_NBUF, _CH, _H), jnp.float32),
        pltpu.VMEM((_CH, _H), jnp.float32),
        pltpu.SemaphoreType.DMA,
        pltpu.SemaphoreType.DMA,
    ],
    compiler_params=pltpu.CompilerParams(
        needs_layout_passes=False, use_tc_tiling_on_sc=False
    ),
)


def _tc_dis_body(hists_ref, dis_ref):
    deg = jnp.sum(hists_ref[...], axis=0) + 1.0
    dis_ref[...] = lax.rsqrt(deg)


_tc_dis = pl.pallas_call(
    _tc_dis_body,
    out_shape=jax.ShapeDtypeStruct((_NPAD,), jnp.float32),
)


def _tc_mm1_body(x_ref, w1_ref, dis_ref, h1_ref, h1t_ref):
    h1 = jnp.dot(x_ref[...], w1_ref[...], preferred_element_type=jnp.float32)
    h1_ref[...] = h1
    h1t_ref[...] = h1 * dis_ref[0:_N]


_tc_mm1 = pl.pallas_call(
    _tc_mm1_body,
    out_shape=[
        jax.ShapeDtypeStruct((_N, _H), jnp.float32),
        jax.ShapeDtypeStruct((_N, _H), jnp.float32),
    ],
)


def _tc_mid_body(dis_ref, acc_ref, h1_ref, b1_ref, w2_ref, h_ref, h2a_ref, h2t_ref):
    dis = dis_ref[0:_N]
    agg = acc_ref[0, 0:_N, :] + acc_ref[1, 0:_N, :]
    hpre = dis * agg + (dis * dis) * h1_ref[...] + b1_ref[...]
    h = jnp.maximum(hpre, 0.0)
    h_ref[...] = h
    h2a = jnp.dot(h, w2_ref[...], preferred_element_type=jnp.float32)
    h2a_ref[...] = h2a
    h2t_ref[...] = h2a * dis


_tc_mid = pl.pallas_call(
    _tc_mid_body,
    out_shape=[
        jax.ShapeDtypeStruct((_N, _H), jnp.float32),
        jax.ShapeDtypeStruct((_N, _H), jnp.float32),
        jax.ShapeDtypeStruct((_N, _H), jnp.float32),
    ],
)


def _tc_out_body(dis_ref, acc_ref, h_ref, h2a_ref, b2_ref, wout_ref, bout_ref, out_ref):
    dis = dis_ref[0:_N]
    agg = acc_ref[0, 0:_N, :] + acc_ref[1, 0:_N, :]
    h2 = dis * agg + (dis * dis) * h2a_ref[...] + b2_ref[...]
    hf = h_ref[...] + h2
    o = jnp.dot(hf, wout_ref[...], preferred_element_type=jnp.float32) + bout_ref[...]
    out_ref[...] = jnp.tanh(o) * 5.0


_tc_out = pl.pallas_call(
    _tc_out_body,
    out_shape=jax.ShapeDtypeStruct((_N, 1), jnp.float32),
)


def kernel(x, edge_index, W1, b1, W2, b2, W_out, b_out):
    pad = _EPAD - _E
    src = jnp.concatenate([edge_index[0], jnp.zeros((pad,), jnp.int32)])
    dst = jnp.concatenate([edge_index[1], jnp.full((pad,), _N + 100, jnp.int32)])
    src = src.reshape(_NWORK * _CPT, _CH)
    dst = dst.reshape(_NWORK * _CPT, _CH)

    hists = _sc_degree(dst)
    dis1d = _tc_dis(hists)
    dis2 = dis1d.reshape(_NPAD, 1)

    h1, h1t = _tc_mm1(x, W1, dis2)
    acc1 = _sc_conv(h1t, src, dst)
    h, h2a, h2t = _tc_mid(dis2, acc1, h1, b1, W2)
    acc2 = _sc_conv(h2t, src, dst)
    return _tc_out(dis2, acc2, h, h2a, b2, W_out, b_out)


# speedup vs baseline: 1.0125x; 1.0125x over previous
import jax
import jax.numpy as jnp
from jax import lax
from jax.experimental import pallas as pl
from jax.experimental.pallas import tpu as pltpu
from jax.experimental.pallas import tpu_sc as plsc

_N = 10000
_NPAD = 10240
_E = 320000
_D = 128
_H = 32
_CH = 128
_NWORK = 32
_CPT = 80
_EPAD = _NWORK * _CPT * _CH
_ROWS_PER_TILE = _NPAD // 16
_NBUF = 12
_GLOOK = 8
_SLAG = _NBUF - _GLOOK

_mesh = plsc.VectorSubcoreMesh(core_axis_name="c", subcore_axis_name="s")


def _sc_degree_body(dst_ref, out_ref, dstv, hist):
    c = lax.axis_index("c")
    s = lax.axis_index("s")
    wid = s * 2 + c
    z16 = jnp.zeros((16,), jnp.float32)

    def zero_body(i, carry):
        hist[pl.ds(i * 16, 16)] = z16
        return carry

    lax.fori_loop(0, _NPAD // 16, zero_body, 0)

    pltpu.sync_copy(dst_ref.at[pl.ds(wid * _CPT, _CPT)], dstv)

    ones = jnp.full((16,), 1.0, jnp.float32)

    def chunk_body(i, carry):
        for j in range(_CH // 16):
            idx = dstv[i, pl.ds(j * 16, 16)]
            plsc.addupdate_scatter(hist, [idx], ones)
        return carry

    lax.fori_loop(0, _CPT, chunk_body, 0)
    pltpu.sync_copy(hist, out_ref.at[wid])


_sc_degree = pl.kernel(
    _sc_degree_body,
    out_type=jax.ShapeDtypeStruct((_NWORK, _NPAD), jnp.float32),
    mesh=_mesh,
    scratch_types=[
        pltpu.VMEM((_CPT, _CH), jnp.int32),
        pltpu.VMEM((_NPAD,), jnp.float32),
    ],
    compiler_params=pltpu.CompilerParams(needs_layout_passes=False),
)


def _sc_conv_body(
    ht_ref, src_ref, dst_ref, out_ref, acc, srcv, dstv, gbuf, zbuf, gsem, ssem
):
    c = lax.axis_index("c")
    s = lax.axis_index("s")
    wid = s * 2 + c
    z16 = jnp.zeros((16,), jnp.float32)

    def zb(i, carry):
        zbuf[i, pl.ds(0, 16)] = z16
        zbuf[i, pl.ds(16, 16)] = z16
        return carry

    lax.fori_loop(0, _CH, zb, 0)
    for k in range(_ROWS_PER_TILE // _CH):
        pltpu.sync_copy(zbuf, acc.at[pl.ds(s * _ROWS_PER_TILE + k * _CH, _CH)])

    pltpu.sync_copy(src_ref.at[pl.ds(wid * _CPT, _CPT)], srcv)
    pltpu.sync_copy(dst_ref.at[pl.ds(wid * _CPT, _CPT)], dstv)
    plsc.subcore_barrier()

    for b in range(_GLOOK):
        pltpu.async_copy(ht_ref.at[srcv.at[b]], gbuf.at[b], gsem)

    def chunk(i, carry):
        buf = lax.rem(i, _NBUF)
        pltpu.make_async_copy(ht_ref.at[srcv.at[i]], gbuf.at[buf], gsem).wait()
        pltpu.async_copy(gbuf.at[buf], acc.at[dstv.at[i]], ssem, add=True)

        @pl.when(i >= _SLAG)
        def _():
            j = i - _SLAG
            jb = lax.rem(j, _NBUF)
            pltpu.make_async_copy(gbuf.at[jb], acc.at[dstv.at[j]], ssem).wait()

        @pl.when(i + _GLOOK < _CPT)
        def _():
            k = i + _GLOOK
            pltpu.async_copy(ht_ref.at[srcv.at[k]], gbuf.at[lax.rem(k, _NBUF)], gsem)

        return carry

    lax.fori_loop(0, _CPT, chunk, 0)
    for j in range(_CPT - _SLAG, _CPT):
        pltpu.make_async_copy(
            gbuf.at[j % _NBUF], acc.at[dstv.at[j]], ssem
        ).wait()
    plsc.subcore_barrier()
    pltpu.sync_copy(
        acc.at[pl.ds(s * _ROWS_PER_TILE, _ROWS_PER_TILE)],
        out_ref.at[c, pl.ds(s * _ROWS_PER_TILE, _ROWS_PER_TILE)],
    )


_sc_conv = pl.kernel(
    _sc_conv_body,
    out_type=jax.ShapeDtypeStruct((2, _NPAD, _H), jnp.float32),
    mesh=_mesh,
    scratch_types=[
        pltpu.VMEM_SHARED((_NPAD, _H), jnp.float32),
        pltpu.VMEM((_CPT, _CH), jnp.int32),
        pltpu.VMEM((_CPT, _CH), jnp.int32),
        pltpu.VMEM((_NBUF, _CH, _H), jnp.float32),
        pltpu.VMEM((_CH, _H), jnp.float32),
        pltpu.SemaphoreType.DMA,
        pltpu.SemaphoreType.DMA,
    ],
    compiler_params=pltpu.CompilerParams(
        needs_layout_passes=False, use_tc_tiling_on_sc=False
    ),
)


def _tc_mm1_body(hists_ref, x_ref, w1_ref, h1_ref, h1t_ref, dis_ref):
    ones = jnp.ones((_NWORK, 1), jnp.float32)
    deg = (
        lax.dot_general(
            hists_ref[...], ones, (((0,), (0,)), ((), ())),
            preferred_element_type=jnp.float32,
        )
        + 1.0
    )
    dis = lax.rsqrt(deg)
    dis_ref[...] = dis
    h1 = jnp.dot(x_ref[...], w1_ref[...], preferred_element_type=jnp.float32)
    h1_ref[...] = h1
    h1t_ref[...] = h1 * dis[0:_N]


_tc_mm1 = pl.pallas_call(
    _tc_mm1_body,
    out_shape=[
        jax.ShapeDtypeStruct((_N, _H), jnp.float32),
        jax.ShapeDtypeStruct((_N, _H), jnp.float32),
        jax.ShapeDtypeStruct((_NPAD, 1), jnp.float32),
    ],
)


def _tc_mid_body(dis_ref, acc_ref, h1_ref, b1_ref, w2_ref, h_ref, h2a_ref, h2t_ref):
    dis = dis_ref[0:_N]
    agg = acc_ref[0, 0:_N, :] + acc_ref[1, 0:_N, :]
    hpre = dis * agg + (dis * dis) * h1_ref[...] + b1_ref[...]
    h = jnp.maximum(hpre, 0.0)
    h_ref[...] = h
    h2a = jnp.dot(h, w2_ref[...], preferred_element_type=jnp.float32)
    h2a_ref[...] = h2a
    h2t_ref[...] = h2a * dis


_tc_mid = pl.pallas_call(
    _tc_mid_body,
    out_shape=[
        jax.ShapeDtypeStruct((_N, _H), jnp.float32),
        jax.ShapeDtypeStruct((_N, _H), jnp.float32),
        jax.ShapeDtypeStruct((_N, _H), jnp.float32),
    ],
)


def _tc_out_body(dis_ref, acc_ref, h_ref, h2a_ref, b2_ref, wout_ref, bout_ref, out_ref):
    dis = dis_ref[0:_N]
    agg = acc_ref[0, 0:_N, :] + acc_ref[1, 0:_N, :]
    h2 = dis * agg + (dis * dis) * h2a_ref[...] + b2_ref[...]
    hf = h_ref[...] + h2
    o = jnp.dot(hf, wout_ref[...], preferred_element_type=jnp.float32) + bout_ref[...]
    out_ref[...] = jnp.tanh(o) * 5.0


_tc_out = pl.pallas_call(
    _tc_out_body,
    out_shape=jax.ShapeDtypeStruct((_N, 1), jnp.float32),
)


def kernel(x, edge_index, W1, b1, W2, b2, W_out, b_out):
    pad = _EPAD - _E
    src = jnp.concatenate([edge_index[0], jnp.zeros((pad,), jnp.int32)])
    dst = jnp.concatenate([edge_index[1], jnp.full((pad,), _N + 100, jnp.int32)])
    src = src.reshape(_NWORK * _CPT, _CH)
    dst = dst.reshape(_NWORK * _CPT, _CH)

    hists = _sc_degree(dst)
    h1, h1t, dis2 = _tc_mm1(hists, x, W1)
    acc1 = _sc_conv(h1t, src, dst)
    h, h2a, h2t = _tc_mid(dis2, acc1, h1, b1, W2)
    acc2 = _sc_conv(h2t, src, dst)
    return _tc_out(dis2, acc2, h, h2a, b2, W_out, b_out)


# speedup vs baseline: 2.1527x; 2.1260x over previous
import jax
import jax.numpy as jnp
from jax import lax
from jax.experimental import pallas as pl
from jax.experimental.pallas import tpu as pltpu
from jax.experimental.pallas import tpu_sc as plsc

_N = 10000
_NPAD = 10240
_E = 320000
_D = 128
_H = 32
_CH = 128
_NWORK = 32
_CPT = 80
_EPAD = _NWORK * _CPT * _CH
_ROWS_PER_TILE = _NPAD // 16
_NBUF = 12
_GLOOK = 8
_SLAG = _NBUF - _GLOOK

_mesh = plsc.VectorSubcoreMesh(core_axis_name="c", subcore_axis_name="s")


def _sc_degree_body(dst_ref, out_ref, dstv, hist):
    c = lax.axis_index("c")
    s = lax.axis_index("s")
    wid = s * 2 + c
    z16 = jnp.zeros((16,), jnp.float32)

    def zero_body(i, carry):
        hist[pl.ds(i * 16, 16)] = z16
        return carry

    lax.fori_loop(0, _NPAD // 16, zero_body, 0)

    pltpu.sync_copy(dst_ref.at[pl.ds(wid * _CPT, _CPT)], dstv)

    ones = jnp.full((16,), 1.0, jnp.float32)

    def chunk_body(i, carry):
        for j in range(_CH // 16):
            idx = dstv[i, pl.ds(j * 16, 16)]
            plsc.addupdate_scatter(hist, [idx], ones)
        return carry

    lax.fori_loop(0, _CPT, chunk_body, 0)
    pltpu.sync_copy(hist, out_ref.at[wid])


_sc_degree = pl.kernel(
    _sc_degree_body,
    out_type=jax.ShapeDtypeStruct((_NWORK, _NPAD), jnp.float32),
    mesh=_mesh,
    scratch_types=[
        pltpu.VMEM((_CPT, _CH), jnp.int32),
        pltpu.VMEM((_NPAD,), jnp.float32),
    ],
    compiler_params=pltpu.CompilerParams(needs_layout_passes=False),
)


def _sc_conv_body(
    ht_ref, src_ref, dst_ref, out_ref, acc, srcv, dstv, gbuf, zbuf, gsem, ssem
):
    c = lax.axis_index("c")
    s = lax.axis_index("s")
    wid = s * 2 + c
    z16 = jnp.zeros((16,), jnp.float32)

    def zb(i, carry):
        zbuf[i, pl.ds(0, 16)] = z16
        zbuf[i, pl.ds(16, 16)] = z16
        return carry

    lax.fori_loop(0, _CH, zb, 0)
    for k in range(_ROWS_PER_TILE // _CH):
        pltpu.sync_copy(zbuf, acc.at[pl.ds(s * _ROWS_PER_TILE + k * _CH, _CH)])

    pltpu.sync_copy(src_ref.at[pl.ds(wid * _CPT, _CPT)], srcv)
    pltpu.sync_copy(dst_ref.at[pl.ds(wid * _CPT, _CPT)], dstv)
    plsc.subcore_barrier()

    for b in range(_GLOOK):
        pltpu.async_copy(ht_ref.at[srcv.at[b]], gbuf.at[b], gsem)

    def chunk(i, carry):
        buf = lax.rem(i, _NBUF)
        pltpu.make_async_copy(ht_ref.at[srcv.at[i]], gbuf.at[buf], gsem).wait()
        pltpu.async_copy(gbuf.at[buf], acc.at[dstv.at[i]], ssem, add=True)

        @pl.when(i >= _SLAG)
        def _():
            j = i - _SLAG
            jb = lax.rem(j, _NBUF)
            pltpu.make_async_copy(gbuf.at[jb], acc.at[dstv.at[j]], ssem).wait()

        @pl.when(i + _GLOOK < _CPT)
        def _():
            k = i + _GLOOK
            pltpu.async_copy(ht_ref.at[srcv.at[k]], gbuf.at[lax.rem(k, _NBUF)], gsem)

        return carry

    lax.fori_loop(0, _CPT, chunk, 0)
    for j in range(_CPT - _SLAG, _CPT):
        pltpu.make_async_copy(
            gbuf.at[j % _NBUF], acc.at[dstv.at[j]], ssem
        ).wait()
    plsc.subcore_barrier()
    pltpu.sync_copy(
        acc.at[pl.ds(s * _ROWS_PER_TILE, _ROWS_PER_TILE)],
        out_ref.at[c, pl.ds(s * _ROWS_PER_TILE, _ROWS_PER_TILE)],
    )


_sc_conv = pl.kernel(
    _sc_conv_body,
    out_type=jax.ShapeDtypeStruct((2, _NPAD, _H), jnp.float32),
    mesh=_mesh,
    scratch_types=[
        pltpu.VMEM_SHARED((_NPAD, _H), jnp.float32),
        pltpu.VMEM((_CPT, _CH), jnp.int32),
        pltpu.VMEM((_CPT, _CH), jnp.int32),
        pltpu.VMEM((_NBUF, _CH, _H), jnp.float32),
        pltpu.VMEM((_CH, _H), jnp.float32),
        pltpu.SemaphoreType.DMA,
        pltpu.SemaphoreType.DMA,
    ],
    compiler_params=pltpu.CompilerParams(
        needs_layout_passes=False, use_tc_tiling_on_sc=False
    ),
)


def _tc_mm1_body(hists_ref, x_ref, w1_ref, h1_ref, h1t_ref, dis_ref):
    ones = jnp.ones((_NWORK, 1), jnp.float32)
    deg = (
        lax.dot_general(
            hists_ref[...], ones, (((0,), (0,)), ((), ())),
            preferred_element_type=jnp.float32,
        )
        + 1.0
    )
    dis = lax.rsqrt(deg)
    dis_ref[...] = dis
    h1 = jnp.dot(x_ref[...], w1_ref[...], preferred_element_type=jnp.float32)
    h1_ref[...] = h1
    h1t_ref[...] = h1 * dis[0:_N]


_tc_mm1 = pl.pallas_call(
    _tc_mm1_body,
    out_shape=[
        jax.ShapeDtypeStruct((_N, _H), jnp.float32),
        jax.ShapeDtypeStruct((_N, _H), jnp.float32),
        jax.ShapeDtypeStruct((_NPAD, 1), jnp.float32),
    ],
)


def _tc_mid_body(dis_ref, acc_ref, h1_ref, b1_ref, w2_ref, h_ref, h2a_ref, h2t_ref):
    dis = dis_ref[0:_N]
    agg = acc_ref[0, 0:_N, :] + acc_ref[1, 0:_N, :]
    hpre = dis * agg + (dis * dis) * h1_ref[...] + b1_ref[...]
    h = jnp.maximum(hpre, 0.0)
    h_ref[...] = h
    h2a = jnp.dot(h, w2_ref[...], preferred_element_type=jnp.float32)
    h2a_ref[...] = h2a
    h2t_ref[...] = h2a * dis


_tc_mid = pl.pallas_call(
    _tc_mid_body,
    out_shape=[
        jax.ShapeDtypeStruct((_N, _H), jnp.float32),
        jax.ShapeDtypeStruct((_N, _H), jnp.float32),
        jax.ShapeDtypeStruct((_N, _H), jnp.float32),
    ],
)


def _tc_out_body(dis_ref, acc_ref, h_ref, h2a_ref, b2_ref, wout_ref, bout_ref, out_ref):
    dis = dis_ref[0:_N]
    agg = acc_ref[0, 0:_N, :] + acc_ref[1, 0:_N, :]
    h2 = dis * agg + (dis * dis) * h2a_ref[...] + b2_ref[...]
    hf = h_ref[...] + h2
    o = jnp.dot(hf, wout_ref[...], preferred_element_type=jnp.float32) + bout_ref[...]
    out_ref[...] = jnp.tanh(o) * 5.0


_tc_out = pl.pallas_call(
    _tc_out_body,
    out_shape=jax.ShapeDtypeStruct((_N, 1), jnp.float32),
)


def kernel(x, edge_index, W1, b1, W2, b2, W_out, b_out):
    pad = _EPAD - _E
    pad_ar = jnp.arange(pad, dtype=jnp.int32)
    src = jnp.concatenate([edge_index[0], pad_ar % _N])
    dst = jnp.concatenate([edge_index[1], _N + pad_ar % (_NPAD - _N)])
    src = src.reshape(_NWORK * _CPT, _CH)
    dst = dst.reshape(_NWORK * _CPT, _CH)

    hists = _sc_degree(dst)
    h1, h1t, dis2 = _tc_mm1(hists, x, W1)
    acc1 = _sc_conv(h1t, src, dst)
    h, h2a, h2t = _tc_mid(dis2, acc1, h1, b1, W2)
    acc2 = _sc_conv(h2t, src, dst)
    return _tc_out(dis2, acc2, h, h2a, b2, W_out, b_out)


# speedup vs baseline: 2.2310x; 1.0364x over previous
import jax
import jax.numpy as jnp
from jax import lax
from jax.experimental import pallas as pl
from jax.experimental.pallas import tpu as pltpu
from jax.experimental.pallas import tpu_sc as plsc

_N = 10000
_NPAD = 10240
_E = 320000
_D = 128
_H = 32
_CH = 128
_NWORK = 32
_CPT = 80
_EPAD = _NWORK * _CPT * _CH
_ROWS_PER_TILE = _NPAD // 16
_NBUF = 12
_GLOOK = 8
_SLAG = _NBUF - _GLOOK

_mesh = plsc.VectorSubcoreMesh(core_axis_name="c", subcore_axis_name="s")


def _sc_degree_body(dst_ref, out_ref, dstv, hist, isem):
    c = lax.axis_index("c")
    s = lax.axis_index("s")
    wid = s * 2 + c
    z16 = jnp.zeros((16,), jnp.float32)

    icp = pltpu.async_copy(dst_ref.at[pl.ds(wid * _CPT, _CPT)], dstv, isem)

    def zero_body(i, carry):
        hist[pl.ds(i * 16, 16)] = z16
        return carry

    lax.fori_loop(0, _NPAD // 16, zero_body, 0)
    icp.wait()

    ones = jnp.full((16,), 1.0, jnp.float32)

    def chunk_body(i, carry):
        for j in range(_CH // 16):
            idx = dstv[i, pl.ds(j * 16, 16)]
            plsc.addupdate_scatter(hist, [idx], ones)
        return carry

    lax.fori_loop(0, _CPT, chunk_body, 0)
    pltpu.sync_copy(hist, out_ref.at[wid])


_sc_degree = pl.kernel(
    _sc_degree_body,
    out_type=jax.ShapeDtypeStruct((_NWORK, _NPAD), jnp.float32),
    mesh=_mesh,
    scratch_types=[
        pltpu.VMEM((_CPT, _CH), jnp.int32),
        pltpu.VMEM((_NPAD,), jnp.float32),
        pltpu.SemaphoreType.DMA,
    ],
    compiler_params=pltpu.CompilerParams(needs_layout_passes=False),
)


def _sc_conv_body(
    ht_ref, src_ref, dst_ref, out_ref, acc, srcv, dstv, gbuf, zbuf, gsem, ssem, isem
):
    c = lax.axis_index("c")
    s = lax.axis_index("s")
    wid = s * 2 + c
    z16 = jnp.zeros((16,), jnp.float32)

    icp1 = pltpu.async_copy(src_ref.at[pl.ds(wid * _CPT, _CPT)], srcv, isem)
    icp2 = pltpu.async_copy(dst_ref.at[pl.ds(wid * _CPT, _CPT)], dstv, isem)

    def zb(i, carry):
        zbuf[i, pl.ds(0, 16)] = z16
        zbuf[i, pl.ds(16, 16)] = z16
        return carry

    lax.fori_loop(0, _CH, zb, 0)
    for k in range(_ROWS_PER_TILE // _CH):
        pltpu.sync_copy(zbuf, acc.at[pl.ds(s * _ROWS_PER_TILE + k * _CH, _CH)])

    icp1.wait()
    icp2.wait()
    plsc.subcore_barrier()

    for b in range(_GLOOK):
        pltpu.async_copy(ht_ref.at[srcv.at[b]], gbuf.at[b], gsem)

    def chunk(i, carry):
        buf = lax.rem(i, _NBUF)
        pltpu.make_async_copy(ht_ref.at[srcv.at[i]], gbuf.at[buf], gsem).wait()
        pltpu.async_copy(gbuf.at[buf], acc.at[dstv.at[i]], ssem, add=True)

        @pl.when(i >= _SLAG)
        def _():
            j = i - _SLAG
            jb = lax.rem(j, _NBUF)
            pltpu.make_async_copy(gbuf.at[jb], acc.at[dstv.at[j]], ssem).wait()

        @pl.when(i + _GLOOK < _CPT)
        def _():
            k = i + _GLOOK
            pltpu.async_copy(ht_ref.at[srcv.at[k]], gbuf.at[lax.rem(k, _NBUF)], gsem)

        return carry

    lax.fori_loop(0, _CPT, chunk, 0)
    for j in range(_CPT - _SLAG, _CPT):
        pltpu.make_async_copy(
            gbuf.at[j % _NBUF], acc.at[dstv.at[j]], ssem
        ).wait()
    plsc.subcore_barrier()
    pltpu.sync_copy(
        acc.at[pl.ds(s * _ROWS_PER_TILE, _ROWS_PER_TILE)],
        out_ref.at[c, pl.ds(s * _ROWS_PER_TILE, _ROWS_PER_TILE)],
    )


_sc_conv = pl.kernel(
    _sc_conv_body,
    out_type=jax.ShapeDtypeStruct((2, _NPAD, _H), jnp.float32),
    mesh=_mesh,
    scratch_types=[
        pltpu.VMEM_SHARED((_NPAD, _H), jnp.float32),
        pltpu.VMEM((_CPT, _CH), jnp.int32),
        pltpu.VMEM((_CPT, _CH), jnp.int32),
        pltpu.VMEM((_NBUF, _CH, _H), jnp.float32),
        pltpu.VMEM((_CH, _H), jnp.float32),
        pltpu.SemaphoreType.DMA,
        pltpu.SemaphoreType.DMA,
        pltpu.SemaphoreType.DMA,
    ],
    compiler_params=pltpu.CompilerParams(
        needs_layout_passes=False, use_tc_tiling_on_sc=False
    ),
)


def _tc_mm1_body(hists_ref, x_ref, w1_ref, h1_ref, h1t_ref, dis_ref):
    ones = jnp.ones((_NWORK, 1), jnp.float32)
    deg = (
        lax.dot_general(
            hists_ref[...], ones, (((0,), (0,)), ((), ())),
            preferred_element_type=jnp.float32,
        )
        + 1.0
    )
    dis = lax.rsqrt(deg)
    dis_ref[...] = dis
    h1 = jnp.dot(x_ref[...], w1_ref[...], preferred_element_type=jnp.float32)
    h1_ref[...] = h1
    h1t_ref[...] = h1 * dis[0:_N]


_tc_mm1 = pl.pallas_call(
    _tc_mm1_body,
    out_shape=[
        jax.ShapeDtypeStruct((_N, _H), jnp.float32),
        jax.ShapeDtypeStruct((_N, _H), jnp.float32),
        jax.ShapeDtypeStruct((_NPAD, 1), jnp.float32),
    ],
)


def _tc_mid_body(dis_ref, acc_ref, h1_ref, b1_ref, w2_ref, h_ref, h2a_ref, h2t_ref):
    dis = dis_ref[0:_N]
    agg = acc_ref[0, 0:_N, :] + acc_ref[1, 0:_N, :]
    hpre = dis * agg + (dis * dis) * h1_ref[...] + b1_ref[...]
    h = jnp.maximum(hpre, 0.0)
    h_ref[...] = h
    h2a = jnp.dot(h, w2_ref[...], preferred_element_type=jnp.float32)
    h2a_ref[...] = h2a
    h2t_ref[...] = h2a * dis


_tc_mid = pl.pallas_call(
    _tc_mid_body,
    out_shape=[
        jax.ShapeDtypeStruct((_N, _H), jnp.float32),
        jax.ShapeDtypeStruct((_N, _H), jnp.float32),
        jax.ShapeDtypeStruct((_N, _H), jnp.float32),
    ],
)


def _tc_out_body(dis_ref, acc_ref, h_ref, h2a_ref, b2_ref, wout_ref, bout_ref, out_ref):
    dis = dis_ref[0:_N]
    agg = acc_ref[0, 0:_N, :] + acc_ref[1, 0:_N, :]
    h2 = dis * agg + (dis * dis) * h2a_ref[...] + b2_ref[...]
    hf = h_ref[...] + h2
    o = jnp.dot(hf, wout_ref[...], preferred_element_type=jnp.float32) + bout_ref[...]
    out_ref[...] = jnp.tanh(o) * 5.0


_tc_out = pl.pallas_call(
    _tc_out_body,
    out_shape=jax.ShapeDtypeStruct((_N, 1), jnp.float32),
)


def kernel(x, edge_index, W1, b1, W2, b2, W_out, b_out):
    pad = _EPAD - _E
    pad_ar = jnp.arange(pad, dtype=jnp.int32)
    src = jnp.concatenate([edge_index[0], pad_ar % _N])
    dst = jnp.concatenate([edge_index[1], _N + pad_ar % (_NPAD - _N)])
    src = src.reshape(_NWORK * _CPT, _CH)
    dst = dst.reshape(_NWORK * _CPT, _CH)

    hists = _sc_degree(dst)
    h1, h1t, dis2 = _tc_mm1(hists, x, W1)
    acc1 = _sc_conv(h1t, src, dst)
    h, h2a, h2t = _tc_mid(dis2, acc1, h1, b1, W2)
    acc2 = _sc_conv(h2t, src, dst)
    return _tc_out(dis2, acc2, h, h2a, b2, W_out, b_out)
